# baseline (speedup 1.0000x reference)
import jax
import jax.numpy as jnp
from jax import lax
from jax.experimental import pallas as pl
from jax.experimental.pallas import tpu as pltpu
from jax.experimental.pallas import tpu_sc as plsc

N = 10000
E = 320000
D = 128
H = 128
G = 64
L = 5

NC = 2
NS = 16
NW = NC * NS

CHUNK = 128
N_PAD = 10240
ROWS_PER_TILE = N_PAD // NS
EPW = -(-E // NW)
NCH = 80
CPW_P = 88
E_PAD = NW * CPW_P * CHUNK

BLK = 1024
NB = N_PAD // BLK


def _proj_body(x_ref, wp_ref, bp_ref, cw_ref, h_ref, m_ref):
    h = jax.nn.relu(jnp.dot(x_ref[...], wp_ref[...],
                            preferred_element_type=jnp.float32) + bp_ref[...])
    h_ref[...] = h
    m_ref[...] = jnp.dot(h, cw_ref[...], preferred_element_type=jnp.float32)


def _proj(x, wp_t, bp, cw0):
    return pl.pallas_call(
        _proj_body,
        grid=(NB,),
        in_specs=[
            pl.BlockSpec((BLK, D), lambda i: (i, 0)),
            pl.BlockSpec((D, H), lambda i: (0, 0)),
            pl.BlockSpec((1, H), lambda i: (0, 0)),
            pl.BlockSpec((H, H), lambda i: (0, 0)),
        ],
        out_specs=[
            pl.BlockSpec((BLK, H), lambda i: (i, 0)),
            pl.BlockSpec((BLK, H), lambda i: (i, 0)),
        ],
        out_shape=[
            jax.ShapeDtypeStruct((N_PAD, H), jnp.float32),
            jax.ShapeDtypeStruct((N_PAD, H), jnp.float32),
        ],
    )(x, wp_t, bp, cw0)


def _gru_body(a0_ref, a1_ref, h_ref, wih_ref, whh_ref, bih_ref, bhh_ref,
              cw_ref, hn_ref, mn_ref):
    agg = a0_ref[...] + a1_ref[...]
    h = h_ref[...]
    gi = jnp.dot(agg, wih_ref[...], preferred_element_type=jnp.float32) + bih_ref[...]
    gh = jnp.dot(h, whh_ref[...], preferred_element_type=jnp.float32) + bhh_ref[...]
    r = jax.nn.sigmoid(gi[:, :H] + gh[:, :H])
    z = jax.nn.sigmoid(gi[:, H:2 * H] + gh[:, H:2 * H])
    n = jnp.tanh(gi[:, 2 * H:] + r * gh[:, 2 * H:])
    hn = (1.0 - z) * n + z * h
    hn_ref[...] = hn
    mn_ref[...] = jnp.dot(hn, cw_ref[...], preferred_element_type=jnp.float32)


def _gru(a0, a1, h, wih_t, whh_t, bih, bhh, cw_next):
    return pl.pallas_call(
        _gru_body,
        grid=(NB,),
        in_specs=[
            pl.BlockSpec((BLK, H), lambda i: (i, 0)),
            pl.BlockSpec((BLK, H), lambda i: (i, 0)),
            pl.BlockSpec((BLK, H), lambda i: (i, 0)),
            pl.BlockSpec((H, 3 * H), lambda i: (0, 0)),
            pl.BlockSpec((H, 3 * H), lambda i: (0, 0)),
            pl.BlockSpec((1, 3 * H), lambda i: (0, 0)),
            pl.BlockSpec((1, 3 * H), lambda i: (0, 0)),
            pl.BlockSpec((H, H), lambda i: (0, 0)),
        ],
        out_specs=[
            pl.BlockSpec((BLK, H), lambda i: (i, 0)),
            pl.BlockSpec((BLK, H), lambda i: (i, 0)),
        ],
        out_shape=[
            jax.ShapeDtypeStruct((N_PAD, H), jnp.float32),
            jax.ShapeDtypeStruct((N_PAD, H), jnp.float32),
        ],
    )(a0, a1, h, wih_t, whh_t, bih, bhh, cw_next)


def _pool_body(h_ref, b_ref, a1_ref, ab1_ref, a2_ref, ab2_ref,
               c1_ref, cb1_ref, c2_ref, cb2_ref, out_ref):
    h = h_ref[...]
    batch = b_ref[...]
    t1 = jnp.tanh(jnp.dot(h, a1_ref[...],
                          preferred_element_type=jnp.float32) + ab1_ref[...])
    s = jnp.dot(t1, a2_ref[...], preferred_element_type=jnp.float32) + ab2_ref[...]
    gids = lax.broadcasted_iota(jnp.int32, (1, G), 1)
    onehot = jnp.where(batch == gids, 1.0, 0.0)
    smax = jnp.max(jnp.where(onehot > 0.0, s, -jnp.inf), axis=0, keepdims=True)
    smax = jnp.maximum(smax, -1e30)
    sm_node = jnp.dot(onehot, smax.T, preferred_element_type=jnp.float32)
    valid = jnp.max(onehot, axis=1, keepdims=True)
    e = jnp.exp(s - sm_node) * valid
    denom = lax.dot_general(onehot, e, (((0,), (0,)), ((), ())),
                            preferred_element_type=jnp.float32)
    denom = jnp.where(denom == 0.0, 1.0, denom)
    inv_node = jnp.dot(onehot, 1.0 / denom, preferred_element_type=jnp.float32)
    w = e * inv_node
    emb = lax.dot_general(onehot, h * w, (((0,), (0,)), ((), ())),
                          preferred_element_type=jnp.float32)
    hid = jax.nn.relu(jnp.dot(emb, c1_ref[...],
                              preferred_element_type=jnp.float32) + cb1_ref[...])
    out_ref[...] = jnp.dot(hid, c2_ref[...],
                           preferred_element_type=jnp.float32) + cb2_ref[...]


def _pool(h, batch2d, a1_t, ab1, a2_t, ab2, c1_t, cb1, c2_t, cb2):
    return pl.pallas_call(
        _pool_body,
        grid=(1,),
        in_specs=[
            pl.BlockSpec((N_PAD, H), lambda i: (0, 0)),
            pl.BlockSpec((N_PAD, 1), lambda i: (0, 0)),
            pl.BlockSpec((H, H // 2), lambda i: (0, 0)),
            pl.BlockSpec((1, H // 2), lambda i: (0, 0)),
            pl.BlockSpec((H // 2, 1), lambda i: (0, 0)),
            pl.BlockSpec((1, 1), lambda i: (0, 0)),
            pl.BlockSpec((H, H // 2), lambda i: (0, 0)),
            pl.BlockSpec((1, H // 2), lambda i: (0, 0)),
            pl.BlockSpec((H // 2, 1), lambda i: (0, 0)),
            pl.BlockSpec((1, 1), lambda i: (0, 0)),
        ],
        out_specs=pl.BlockSpec((G, 1), lambda i: (0, 0)),
        out_shape=jax.ShapeDtypeStruct((G, 1), jnp.float32),
    )(h, batch2d, a1_t, ab1, a2_t, ab2, c1_t, cb1, c2_t, cb2)


ZROWS = 16
NCH_H = NCH // 2
IDXC = 48


def _agg_body(m_hbm, src_hbm, dst_hbm, agg_out, src_all, dst_all,
              rows0, rows1, zbuf, agg_sh,
              sem_i, sem_z, sem_g0, sem_g1, sem_s0, sem_s1):
    c = lax.axis_index("c")
    s = lax.axis_index("s")
    base = s * ROWS_PER_TILE
    rows_b = (rows0, rows1)
    sem_g = (sem_g0, sem_g1)
    sem_s = (sem_s0, sem_s1)

    pltpu.async_copy(src_hbm.at[c, s, pl.ds(0, IDXC)], src_all, sem_i)
    pltpu.async_copy(dst_hbm.at[c, s, pl.ds(0, IDXC)], dst_all, sem_i)

    zero = jnp.zeros((16,), jnp.float32)
    for i in range(ZROWS):
        for j in range(H // 16):
            zbuf[i, pl.ds(j * 16, 16)] = zero

    for k in range(ROWS_PER_TILE // ZROWS):
        pltpu.async_copy(zbuf, agg_sh.at[pl.ds(base + k * ZROWS, ZROWS)],
                         sem_z)
    for k in range(ROWS_PER_TILE // ZROWS):
        pltpu.make_async_copy(zbuf, agg_sh.at[pl.ds(base + k * ZROWS, ZROWS)],
                              sem_z).wait()
    pltpu.make_async_copy(src_hbm.at[c, s, pl.ds(0, IDXC)], src_all,
                          sem_i).wait()
    pltpu.make_async_copy(dst_hbm.at[c, s, pl.ds(0, IDXC)], dst_all,
                          sem_i).wait()
    plsc.subcore_barrier()

    def _half(hh, carry):
        pltpu.async_copy(m_hbm.at[src_all.at[0]], rows0, sem_g0)
        pltpu.async_copy(m_hbm.at[src_all.at[1]], rows1, sem_g1)

        def _pair(i, carry2):
            j0 = 2 * i
            js = (j0, j0 + 1)
            for b in range(2):
                pltpu.make_async_copy(m_hbm.at[src_all.at[js[b]]], rows_b[b],
                                      sem_g[b]).wait()
                pltpu.async_copy(rows_b[b], agg_sh.at[dst_all.at[js[b]]],
                                 sem_s[b], add=True)
            for b in range(2):
                pltpu.make_async_copy(rows_b[b], agg_sh.at[dst_all.at[js[b]]],
                                      sem_s[b]).wait()
                pltpu.async_copy(m_hbm.at[src_all.at[js[b] + 2]], rows_b[b],
                                 sem_g[b])
            return carry2

        lax.fori_loop(0, NCH_H // 2, _pair, 0)
        pltpu.make_async_copy(m_hbm.at[src_all.at[NCH_H]], rows0,
                              sem_g0).wait()
        pltpu.make_async_copy(m_hbm.at[src_all.at[NCH_H + 1]], rows1,
                              sem_g1).wait()
        nxt = jnp.minimum((hh + 1) * NCH_H, CPW_P - IDXC)
        pltpu.sync_copy(src_hbm.at[c, s, pl.ds(nxt, IDXC)], src_all)
        pltpu.sync_copy(dst_hbm.at[c, s, pl.ds(nxt, IDXC)], dst_all)
        return carry

    lax.fori_loop(0, 2, _half, 0)
    plsc.subcore_barrier()

    pltpu.sync_copy(agg_sh.at[pl.ds(base, ROWS_PER_TILE)],
                    agg_out.at[c, pl.ds(base, ROWS_PER_TILE)])


_sc_agg_cache = []


def _sc_agg(m, src_p, dst_p):
    if not _sc_agg_cache:
        _sc_agg_cache.append(pl.kernel(
            _agg_body,
            out_type=jax.ShapeDtypeStruct((NC, N_PAD, H), jnp.float32),
            mesh=plsc.VectorSubcoreMesh(core_axis_name="c",
                                        subcore_axis_name="s",
                                        num_cores=NC, num_subcores=NS),
            scratch_types=[
                pltpu.VMEM((IDXC, CHUNK), jnp.int32),
                pltpu.VMEM((IDXC, CHUNK), jnp.int32),
                pltpu.VMEM((CHUNK, H), jnp.float32),
                pltpu.VMEM((CHUNK, H), jnp.float32),
                pltpu.VMEM((ZROWS, H), jnp.float32),
                pltpu.VMEM_SHARED((N_PAD, H), jnp.float32),
                pltpu.SemaphoreType.DMA,
                pltpu.SemaphoreType.DMA,
                pltpu.SemaphoreType.DMA,
                pltpu.SemaphoreType.DMA,
                pltpu.SemaphoreType.DMA,
                pltpu.SemaphoreType.DMA,
            ],
        ))
    return _sc_agg_cache[0](m, src_p, dst_p)


@jax.jit
def _run(x, edge_index, batch, W_proj, b_proj, conv_w, gru_w_ih, gru_w_hh,
         gru_b_ih, gru_b_hh, att_w1, att_b1, att_w2, att_b2, cls_w1, cls_b1,
         cls_w2, cls_b2):
    x_p = jnp.pad(x, ((0, N_PAD - N), (0, 0)))
    src = edge_index[0].astype(jnp.int32)
    dst = edge_index[1].astype(jnp.int32)
    e_scat = NW * NCH * CHUNK
    src_p = jnp.pad(src, (0, e_scat - E)).reshape(NC, NS, NCH, CHUNK)
    dst_p = jnp.pad(dst, (0, e_scat - E),
                    constant_values=N).reshape(NC, NS, NCH, CHUNK)
    src_p = jnp.concatenate(
        [src_p, jnp.zeros((NC, NS, CPW_P - NCH, CHUNK), jnp.int32)], axis=2)
    dst_p = jnp.concatenate(
        [dst_p, jnp.full((NC, NS, CPW_P - NCH, CHUNK), N, jnp.int32)],
        axis=2)
    batch_p = jnp.pad(batch.astype(jnp.int32), (0, N_PAD - N),
                      constant_values=G).reshape(N_PAD, 1)

    wp_t = W_proj.T
    bp = b_proj.reshape(1, H)
    wih_t = gru_w_ih.T
    whh_t = gru_w_hh.T
    bih = gru_b_ih.reshape(1, 3 * H)
    bhh = gru_b_hh.reshape(1, 3 * H)

    h, m = _proj(x_p, wp_t, bp, conv_w[0])
    for i in range(L):
        agg = _sc_agg(m, src_p, dst_p)
        cw_next = conv_w[i + 1] if i + 1 < L else conv_w[0]
        h, m = _gru(agg[0], agg[1], h, wih_t, whh_t, bih, bhh, cw_next)

    out = _pool(h, batch_p, att_w1.T, att_b1.reshape(1, H // 2),
                att_w2.T, att_b2.reshape(1, 1), cls_w1.T,
                cls_b1.reshape(1, H // 2), cls_w2.T, cls_b2.reshape(1, 1))
    return out[:, 0]


def kernel(x, edge_index, batch, W_proj, b_proj, conv_w, gru_w_ih, gru_w_hh,
           gru_b_ih, gru_b_hh, att_w1, att_b1, att_w2, att_b2, cls_w1, cls_b1,
           cls_w2, cls_b2):
    return _run(x, edge_index, batch, W_proj, b_proj, conv_w, gru_w_ih,
                gru_w_hh, gru_b_ih, gru_b_hh, att_w1, att_b1, att_w2, att_b2,
                cls_w1, cls_b1, cls_w2, cls_b2)


# speedup vs baseline: 2.0206x; 2.0206x over previous
import jax
import jax.numpy as jnp
from jax import lax
from jax.experimental import pallas as pl
from jax.experimental.pallas import tpu as pltpu
from jax.experimental.pallas import tpu_sc as plsc

N = 10000
E = 320000
D = 128
H = 128
G = 64
L = 5

NC = 2
NS = 16
NW = NC * NS

CHUNK = 128
N_PAD = 10240
ROWS_PER_TILE = N_PAD // NS
EPW = -(-E // NW)
NCH = -(-EPW // CHUNK)

BLK = 1024
NB = N_PAD // BLK


def _proj_body(x_ref, wp_ref, bp_ref, cw_ref, h_ref, m_ref):
    h = jax.nn.relu(jnp.dot(x_ref[...], wp_ref[...],
                            preferred_element_type=jnp.float32, precision=lax.Precision.HIGHEST) + bp_ref[...])
    h_ref[...] = h
    m_ref[...] = jnp.dot(h, cw_ref[...], preferred_element_type=jnp.float32, precision=lax.Precision.HIGHEST)


def _proj(x, wp_t, bp, cw0):
    return pl.pallas_call(
        _proj_body,
        grid=(NB,),
        in_specs=[
            pl.BlockSpec((BLK, D), lambda i: (i, 0)),
            pl.BlockSpec((D, H), lambda i: (0, 0)),
            pl.BlockSpec((1, H), lambda i: (0, 0)),
            pl.BlockSpec((H, H), lambda i: (0, 0)),
        ],
        out_specs=[
            pl.BlockSpec((BLK, H), lambda i: (i, 0)),
            pl.BlockSpec((BLK, H), lambda i: (i, 0)),
        ],
        out_shape=[
            jax.ShapeDtypeStruct((N_PAD, H), jnp.float32),
            jax.ShapeDtypeStruct((N_PAD, H), jnp.float32),
        ],
    )(x, wp_t, bp, cw0)


def _gru_body(a0_ref, a1_ref, h_ref, wih_ref, whh_ref, bih_ref, bhh_ref,
              cw_ref, hn_ref, mn_ref):
    agg = a0_ref[...] + a1_ref[...]
    h = h_ref[...]
    gi = jnp.dot(agg, wih_ref[...], preferred_element_type=jnp.float32, precision=lax.Precision.HIGHEST) + bih_ref[...]
    gh = jnp.dot(h, whh_ref[...], preferred_element_type=jnp.float32, precision=lax.Precision.HIGHEST) + bhh_ref[...]
    r = jax.nn.sigmoid(gi[:, :H] + gh[:, :H])
    z = jax.nn.sigmoid(gi[:, H:2 * H] + gh[:, H:2 * H])
    n = jnp.tanh(gi[:, 2 * H:] + r * gh[:, 2 * H:])
    hn = (1.0 - z) * n + z * h
    hn_ref[...] = hn
    mn_ref[...] = jnp.dot(hn, cw_ref[...], preferred_element_type=jnp.float32, precision=lax.Precision.HIGHEST)


def _gru(a0, a1, h, wih_t, whh_t, bih, bhh, cw_next):
    return pl.pallas_call(
        _gru_body,
        grid=(NB,),
        in_specs=[
            pl.BlockSpec((BLK, H), lambda i: (i, 0)),
            pl.BlockSpec((BLK, H), lambda i: (i, 0)),
            pl.BlockSpec((BLK, H), lambda i: (i, 0)),
            pl.BlockSpec((H, 3 * H), lambda i: (0, 0)),
            pl.BlockSpec((H, 3 * H), lambda i: (0, 0)),
            pl.BlockSpec((1, 3 * H), lambda i: (0, 0)),
            pl.BlockSpec((1, 3 * H), lambda i: (0, 0)),
            pl.BlockSpec((H, H), lambda i: (0, 0)),
        ],
        out_specs=[
            pl.BlockSpec((BLK, H), lambda i: (i, 0)),
            pl.BlockSpec((BLK, H), lambda i: (i, 0)),
        ],
        out_shape=[
            jax.ShapeDtypeStruct((N_PAD, H), jnp.float32),
            jax.ShapeDtypeStruct((N_PAD, H), jnp.float32),
        ],
    )(a0, a1, h, wih_t, whh_t, bih, bhh, cw_next)


def _pool_body(h_ref, b_ref, a1_ref, ab1_ref, a2_ref, ab2_ref,
               c1_ref, cb1_ref, c2_ref, cb2_ref, out_ref):
    h = h_ref[...]
    batch = b_ref[...]
    t1 = jnp.tanh(jnp.dot(h, a1_ref[...],
                          preferred_element_type=jnp.float32, precision=lax.Precision.HIGHEST) + ab1_ref[...])
    s = jnp.dot(t1, a2_ref[...], preferred_element_type=jnp.float32, precision=lax.Precision.HIGHEST) + ab2_ref[...]
    gids = lax.broadcasted_iota(jnp.int32, (1, G), 1)
    onehot = jnp.where(batch == gids, 1.0, 0.0)
    smax = jnp.max(jnp.where(onehot > 0.0, s, -jnp.inf), axis=0, keepdims=True)
    smax = jnp.maximum(smax, -1e30)
    sm_node = jnp.dot(onehot, smax.T, preferred_element_type=jnp.float32, precision=lax.Precision.HIGHEST)
    valid = jnp.max(onehot, axis=1, keepdims=True)
    e = jnp.exp(s - sm_node) * valid
    denom = lax.dot_general(onehot, e, (((0,), (0,)), ((), ())),
                            preferred_element_type=jnp.float32, precision=lax.Precision.HIGHEST)
    denom = jnp.where(denom == 0.0, 1.0, denom)
    inv_node = jnp.dot(onehot, 1.0 / denom, preferred_element_type=jnp.float32, precision=lax.Precision.HIGHEST)
    w = e * inv_node
    emb = lax.dot_general(onehot, h * w, (((0,), (0,)), ((), ())),
                          preferred_element_type=jnp.float32, precision=lax.Precision.HIGHEST)
    hid = jax.nn.relu(jnp.dot(emb, c1_ref[...],
                              preferred_element_type=jnp.float32, precision=lax.Precision.HIGHEST) + cb1_ref[...])
    out_ref[...] = jnp.dot(hid, c2_ref[...],
                           preferred_element_type=jnp.float32, precision=lax.Precision.HIGHEST) + cb2_ref[...]


def _pool(h, batch2d, a1_t, ab1, a2_t, ab2, c1_t, cb1, c2_t, cb2):
    return pl.pallas_call(
        _pool_body,
        grid=(1,),
        in_specs=[
            pl.BlockSpec((N_PAD, H), lambda i: (0, 0)),
            pl.BlockSpec((N_PAD, 1), lambda i: (0, 0)),
            pl.BlockSpec((H, H // 2), lambda i: (0, 0)),
            pl.BlockSpec((1, H // 2), lambda i: (0, 0)),
            pl.BlockSpec((H // 2, 1), lambda i: (0, 0)),
            pl.BlockSpec((1, 1), lambda i: (0, 0)),
            pl.BlockSpec((H, H // 2), lambda i: (0, 0)),
            pl.BlockSpec((1, H // 2), lambda i: (0, 0)),
            pl.BlockSpec((H // 2, 1), lambda i: (0, 0)),
            pl.BlockSpec((1, 1), lambda i: (0, 0)),
        ],
        out_specs=pl.BlockSpec((G, 1), lambda i: (0, 0)),
        out_shape=jax.ShapeDtypeStruct((G, 1), jnp.float32),
    )(h, batch2d, a1_t, ab1, a2_t, ab2, c1_t, cb1, c2_t, cb2)


ZROWS = 16


def _agg_body(m_hbm, src_hbm, dst_hbm, agg_out, src_all, dst_all, rows,
              zbuf, agg_sh, sem_i, sem_z, sem):
    c = lax.axis_index("c")
    s = lax.axis_index("s")
    base = s * ROWS_PER_TILE

    pltpu.async_copy(src_hbm.at[c, s], src_all, sem_i)
    pltpu.async_copy(dst_hbm.at[c, s], dst_all, sem_i)

    zero = jnp.zeros((16,), jnp.float32)
    for i in range(ZROWS):
        for j in range(H // 16):
            zbuf[i, pl.ds(j * 16, 16)] = zero

    for k in range(ROWS_PER_TILE // ZROWS):
        pltpu.async_copy(zbuf, agg_sh.at[pl.ds(base + k * ZROWS, ZROWS)],
                         sem_z)
    for k in range(ROWS_PER_TILE // ZROWS):
        pltpu.make_async_copy(zbuf, agg_sh.at[pl.ds(base + k * ZROWS, ZROWS)],
                              sem_z).wait()
    pltpu.make_async_copy(src_hbm.at[c, s], src_all, sem_i).wait()
    pltpu.make_async_copy(dst_hbm.at[c, s], dst_all, sem_i).wait()
    plsc.subcore_barrier()

    def _step(j, carry):
        pltpu.async_copy(m_hbm.at[src_all.at[j]], rows, sem).wait()
        pltpu.sync_copy(rows, agg_sh.at[dst_all.at[j]], add=True)
        return carry

    lax.fori_loop(0, NCH, _step, 0)
    plsc.subcore_barrier()

    pltpu.sync_copy(agg_sh.at[pl.ds(base, ROWS_PER_TILE)],
                    agg_out.at[c, pl.ds(base, ROWS_PER_TILE)])


_sc_agg_cache = []


def _sc_agg(m, src_p, dst_p):
    if not _sc_agg_cache:
        _sc_agg_cache.append(pl.kernel(
            _agg_body,
            out_type=jax.ShapeDtypeStruct((NC, N_PAD, H), jnp.float32),
            mesh=plsc.VectorSubcoreMesh(core_axis_name="c",
                                        subcore_axis_name="s",
                                        num_cores=NC, num_subcores=NS),
            scratch_types=[
                pltpu.VMEM((NCH, CHUNK), jnp.int32),
                pltpu.VMEM((NCH, CHUNK), jnp.int32),
                pltpu.VMEM((CHUNK, H), jnp.float32),
                pltpu.VMEM((ZROWS, H), jnp.float32),
                pltpu.VMEM_SHARED((N_PAD, H), jnp.float32),
                pltpu.SemaphoreType.DMA,
                pltpu.SemaphoreType.DMA,
                pltpu.SemaphoreType.DMA,
            ],
        ))
    return _sc_agg_cache[0](m, src_p, dst_p)


@jax.jit
def _run(x, edge_index, batch, W_proj, b_proj, conv_w, gru_w_ih, gru_w_hh,
         gru_b_ih, gru_b_hh, att_w1, att_b1, att_w2, att_b2, cls_w1, cls_b1,
         cls_w2, cls_b2):
    x_p = jnp.pad(x, ((0, N_PAD - N), (0, 0)))
    src = edge_index[0].astype(jnp.int32)
    dst = edge_index[1].astype(jnp.int32)
    e_scat = NW * NCH * CHUNK
    src_p = jnp.pad(src, (0, e_scat - E)).reshape(NC, NS, NCH, CHUNK)
    dst_p = jnp.pad(dst, (0, e_scat - E),
                    constant_values=N).reshape(NC, NS, NCH, CHUNK)
    batch_p = jnp.pad(batch.astype(jnp.int32), (0, N_PAD - N),
                      constant_values=G).reshape(N_PAD, 1)

    wp_t = W_proj.T
    bp = b_proj.reshape(1, H)
    wih_t = gru_w_ih.T
    whh_t = gru_w_hh.T
    bih = gru_b_ih.reshape(1, 3 * H)
    bhh = gru_b_hh.reshape(1, 3 * H)

    h, m = _proj(x_p, wp_t, bp, conv_w[0])
    for i in range(L):
        agg = _sc_agg(m, src_p, dst_p)
        cw_next = conv_w[i + 1] if i + 1 < L else conv_w[0]
        h, m = _gru(agg[0], agg[1], h, wih_t, whh_t, bih, bhh, cw_next)

    out = _pool(h, batch_p, att_w1.T, att_b1.reshape(1, H // 2),
                att_w2.T, att_b2.reshape(1, 1), cls_w1.T,
                cls_b1.reshape(1, H // 2), cls_w2.T, cls_b2.reshape(1, 1))
    return out[:, 0]


def kernel(x, edge_index, batch, W_proj, b_proj, conv_w, gru_w_ih, gru_w_hh,
           gru_b_ih, gru_b_hh, att_w1, att_b1, att_w2, att_b2, cls_w1, cls_b1,
           cls_w2, cls_b2):
    return _run(x, edge_index, batch, W_proj, b_proj, conv_w, gru_w_ih,
                gru_w_hh, gru_b_ih, gru_b_hh, att_w1, att_b1, att_w2, att_b2,
                cls_w1, cls_b1, cls_w2, cls_b2)


# speedup vs baseline: 2.1947x; 1.0862x over previous
import jax
import jax.numpy as jnp
from jax import lax
from jax.experimental import pallas as pl
from jax.experimental.pallas import tpu as pltpu
from jax.experimental.pallas import tpu_sc as plsc

N = 10000
E = 320000
D = 128
H = 128
G = 64
L = 5

NC = 2
NS = 16
NW = NC * NS

CHUNK = 128
N_PAD = 10240
ROWS_PER_TILE = N_PAD // NS
EPW = -(-E // NW)
NCH = -(-EPW // CHUNK)

BLK = 1024
NB = N_PAD // BLK


def _proj_body(x_ref, wp_ref, bp_ref, cw_ref, h_ref, m_ref):
    h = jax.nn.relu(jnp.dot(x_ref[...], wp_ref[...],
                            preferred_element_type=jnp.float32) + bp_ref[...])
    h_ref[...] = h
    m_ref[...] = jnp.dot(h, cw_ref[...], preferred_element_type=jnp.float32)


def _proj(x, wp_t, bp, cw0):
    return pl.pallas_call(
        _proj_body,
        grid=(NB,),
        in_specs=[
            pl.BlockSpec((BLK, D), lambda i: (i, 0)),
            pl.BlockSpec((D, H), lambda i: (0, 0)),
            pl.BlockSpec((1, H), lambda i: (0, 0)),
            pl.BlockSpec((H, H), lambda i: (0, 0)),
        ],
        out_specs=[
            pl.BlockSpec((BLK, H), lambda i: (i, 0)),
            pl.BlockSpec((BLK, H), lambda i: (i, 0)),
        ],
        out_shape=[
            jax.ShapeDtypeStruct((N_PAD, H), jnp.float32),
            jax.ShapeDtypeStruct((N_PAD, H), jnp.float32),
        ],
    )(x, wp_t, bp, cw0)


def _gru_body(a0_ref, a1_ref, h_ref, wih_ref, whh_ref, bih_ref, bhh_ref,
              cw_ref, hn_ref, mn_ref):
    agg = a0_ref[...] + a1_ref[...]
    h = h_ref[...]
    gi = jnp.dot(agg, wih_ref[...], preferred_element_type=jnp.float32) + bih_ref[...]
    gh = jnp.dot(h, whh_ref[...], preferred_element_type=jnp.float32) + bhh_ref[...]
    r = jax.nn.sigmoid(gi[:, :H] + gh[:, :H])
    z = jax.nn.sigmoid(gi[:, H:2 * H] + gh[:, H:2 * H])
    n = jnp.tanh(gi[:, 2 * H:] + r * gh[:, 2 * H:])
    hn = (1.0 - z) * n + z * h
    hn_ref[...] = hn
    mn_ref[...] = jnp.dot(hn, cw_ref[...], preferred_element_type=jnp.float32)


def _gru(a0, a1, h, wih_t, whh_t, bih, bhh, cw_next):
    return pl.pallas_call(
        _gru_body,
        grid=(NB,),
        in_specs=[
            pl.BlockSpec((BLK, H), lambda i: (i, 0)),
            pl.BlockSpec((BLK, H), lambda i: (i, 0)),
            pl.BlockSpec((BLK, H), lambda i: (i, 0)),
            pl.BlockSpec((H, 3 * H), lambda i: (0, 0)),
            pl.BlockSpec((H, 3 * H), lambda i: (0, 0)),
            pl.BlockSpec((1, 3 * H), lambda i: (0, 0)),
            pl.BlockSpec((1, 3 * H), lambda i: (0, 0)),
            pl.BlockSpec((H, H), lambda i: (0, 0)),
        ],
        out_specs=[
            pl.BlockSpec((BLK, H), lambda i: (i, 0)),
            pl.BlockSpec((BLK, H), lambda i: (i, 0)),
        ],
        out_shape=[
            jax.ShapeDtypeStruct((N_PAD, H), jnp.float32),
            jax.ShapeDtypeStruct((N_PAD, H), jnp.float32),
        ],
    )(a0, a1, h, wih_t, whh_t, bih, bhh, cw_next)


def _pool_body(h_ref, b_ref, a1_ref, ab1_ref, a2_ref, ab2_ref,
               c1_ref, cb1_ref, c2_ref, cb2_ref, out_ref):
    h = h_ref[...]
    batch = b_ref[...]
    t1 = jnp.tanh(jnp.dot(h, a1_ref[...],
                          preferred_element_type=jnp.float32) + ab1_ref[...])
    s = jnp.dot(t1, a2_ref[...], preferred_element_type=jnp.float32) + ab2_ref[...]
    gids = lax.broadcasted_iota(jnp.int32, (1, G), 1)
    onehot = jnp.where(batch == gids, 1.0, 0.0)
    smax = jnp.max(jnp.where(onehot > 0.0, s, -jnp.inf), axis=0, keepdims=True)
    smax = jnp.maximum(smax, -1e30)
    sm_node = jnp.dot(onehot, smax.T, preferred_element_type=jnp.float32)
    valid = jnp.max(onehot, axis=1, keepdims=True)
    e = jnp.exp(s - sm_node) * valid
    denom = lax.dot_general(onehot, e, (((0,), (0,)), ((), ())),
                            preferred_element_type=jnp.float32)
    denom = jnp.where(denom == 0.0, 1.0, denom)
    inv_node = jnp.dot(onehot, 1.0 / denom, preferred_element_type=jnp.float32)
    w = e * inv_node
    emb = lax.dot_general(onehot, h * w, (((0,), (0,)), ((), ())),
                          preferred_element_type=jnp.float32)
    hid = jax.nn.relu(jnp.dot(emb, c1_ref[...],
                              preferred_element_type=jnp.float32) + cb1_ref[...])
    out_ref[...] = jnp.dot(hid, c2_ref[...],
                           preferred_element_type=jnp.float32) + cb2_ref[...]


def _pool(h, batch2d, a1_t, ab1, a2_t, ab2, c1_t, cb1, c2_t, cb2):
    return pl.pallas_call(
        _pool_body,
        grid=(1,),
        in_specs=[
            pl.BlockSpec((N_PAD, H), lambda i: (0, 0)),
            pl.BlockSpec((N_PAD, 1), lambda i: (0, 0)),
            pl.BlockSpec((H, H // 2), lambda i: (0, 0)),
            pl.BlockSpec((1, H // 2), lambda i: (0, 0)),
            pl.BlockSpec((H // 2, 1), lambda i: (0, 0)),
            pl.BlockSpec((1, 1), lambda i: (0, 0)),
            pl.BlockSpec((H, H // 2), lambda i: (0, 0)),
            pl.BlockSpec((1, H // 2), lambda i: (0, 0)),
            pl.BlockSpec((H // 2, 1), lambda i: (0, 0)),
            pl.BlockSpec((1, 1), lambda i: (0, 0)),
        ],
        out_specs=pl.BlockSpec((G, 1), lambda i: (0, 0)),
        out_shape=jax.ShapeDtypeStruct((G, 1), jnp.float32),
    )(h, batch2d, a1_t, ab1, a2_t, ab2, c1_t, cb1, c2_t, cb2)


ZROWS = 16


def _agg_body(m_hbm, src_hbm, dst_hbm, agg_out, src_all, dst_all, idx_d,
              rows, zbuf, agg_sh, sem_i, sem_z, sem):
    c = lax.axis_index("c")
    s = lax.axis_index("s")
    base = s * ROWS_PER_TILE

    pltpu.async_copy(src_hbm.at[c, s], src_all, sem_i)
    pltpu.async_copy(dst_hbm.at[c, s], dst_all, sem_i)

    zero = jnp.zeros((16,), jnp.float32)
    for i in range(ZROWS):
        for j in range(H // 16):
            zbuf[i, pl.ds(j * 16, 16)] = zero

    for k in range(ROWS_PER_TILE // ZROWS):
        pltpu.async_copy(zbuf, agg_sh.at[pl.ds(base + k * ZROWS, ZROWS)],
                         sem_z)
    for k in range(ROWS_PER_TILE // ZROWS):
        pltpu.make_async_copy(zbuf, agg_sh.at[pl.ds(base + k * ZROWS, ZROWS)],
                              sem_z).wait()
    pltpu.make_async_copy(src_hbm.at[c, s], src_all, sem_i).wait()
    pltpu.make_async_copy(dst_hbm.at[c, s], dst_all, sem_i).wait()
    plsc.subcore_barrier()

    def _step(j, carry):
        for k in range(CHUNK // 16):
            idx_d[pl.ds(k * 16, 16)] = dst_all[j, pl.ds(k * 16, 16)]
        pltpu.async_copy(m_hbm.at[src_all.at[j]], rows, sem).wait()
        pltpu.sync_copy(rows, agg_sh.at[idx_d], add=True)
        return carry

    lax.fori_loop(0, NCH, _step, 0)
    plsc.subcore_barrier()

    pltpu.sync_copy(agg_sh.at[pl.ds(base, ROWS_PER_TILE)],
                    agg_out.at[c, pl.ds(base, ROWS_PER_TILE)])


_sc_agg_cache = []


def _sc_agg(m, src_p, dst_p):
    if not _sc_agg_cache:
        _sc_agg_cache.append(pl.kernel(
            _agg_body,
            out_type=jax.ShapeDtypeStruct((NC, N_PAD, H), jnp.float32),
            mesh=plsc.VectorSubcoreMesh(core_axis_name="c",
                                        subcore_axis_name="s",
                                        num_cores=NC, num_subcores=NS),
            scratch_types=[
                pltpu.VMEM((NCH, CHUNK), jnp.int32),
                pltpu.VMEM((NCH, CHUNK), jnp.int32),
                pltpu.VMEM((CHUNK,), jnp.int32),
                pltpu.VMEM((CHUNK, H), jnp.float32),
                pltpu.VMEM((ZROWS, H), jnp.float32),
                pltpu.VMEM_SHARED((N_PAD, H), jnp.float32),
                pltpu.SemaphoreType.DMA,
                pltpu.SemaphoreType.DMA,
                pltpu.SemaphoreType.DMA,
            ],
        ))
    return _sc_agg_cache[0](m, src_p, dst_p)


@jax.jit
def _run(x, edge_index, batch, W_proj, b_proj, conv_w, gru_w_ih, gru_w_hh,
         gru_b_ih, gru_b_hh, att_w1, att_b1, att_w2, att_b2, cls_w1, cls_b1,
         cls_w2, cls_b2):
    x_p = jnp.pad(x, ((0, N_PAD - N), (0, 0)))
    src = edge_index[0].astype(jnp.int32)
    dst = edge_index[1].astype(jnp.int32)
    e_scat = NW * NCH * CHUNK
    src_p = jnp.pad(src, (0, e_scat - E)).reshape(NC, NS, NCH, CHUNK)
    dst_p = jnp.pad(dst, (0, e_scat - E),
                    constant_values=N).reshape(NC, NS, NCH, CHUNK)
    batch_p = jnp.pad(batch.astype(jnp.int32), (0, N_PAD - N),
                      constant_values=G).reshape(N_PAD, 1)

    wp_t = W_proj.T
    bp = b_proj.reshape(1, H)
    wih_t = gru_w_ih.T
    whh_t = gru_w_hh.T
    bih = gru_b_ih.reshape(1, 3 * H)
    bhh = gru_b_hh.reshape(1, 3 * H)

    h, m = _proj(x_p, wp_t, bp, conv_w[0])
    for i in range(L):
        agg = _sc_agg(m, src_p, dst_p)
        cw_next = conv_w[i + 1] if i + 1 < L else conv_w[0]
        h, m = _gru(agg[0], agg[1], h, wih_t, whh_t, bih, bhh, cw_next)

    out = _pool(h, batch_p, att_w1.T, att_b1.reshape(1, H // 2),
                att_w2.T, att_b2.reshape(1, 1), cls_w1.T,
                cls_b1.reshape(1, H // 2), cls_w2.T, cls_b2.reshape(1, 1))
    return out[:, 0]


def kernel(x, edge_index, batch, W_proj, b_proj, conv_w, gru_w_ih, gru_w_hh,
           gru_b_ih, gru_b_hh, att_w1, att_b1, att_w2, att_b2, cls_w1, cls_b1,
           cls_w2, cls_b2):
    return _run(x, edge_index, batch, W_proj, b_proj, conv_w, gru_w_ih,
                gru_w_hh, gru_b_ih, gru_b_hh, att_w1, att_b1, att_w2, att_b2,
                cls_w1, cls_b1, cls_w2, cls_b2)
